# baseline (speedup 1.0000x reference)
import jax
import jax.numpy as jnp
from jax.experimental import pallas as pl
from jax.experimental.pallas import tpu as pltpu

SB_EPSILON = 0.1
EPS4 = SB_EPSILON / 4.0
B, S, L = 4, 512, 16
SL = S * L
ROWS = 64
NC = S // ROWS


def _body(p_ref, t_ref, m_ref, tp_ref, tn_ref, mp_ref, mn_ref,
          loss_ref, cnt_ref, c_ref):
    b = pl.program_id(0)
    c = pl.program_id(1)
    pv = jnp.where(c > 0, 1.0, 0.0).astype(jnp.float32)
    nv = jnp.where(c < NC - 1, 1.0, 0.0).astype(jnp.float32)

    p = p_ref[0]
    t = t_ref[0]
    maskv = m_ref[0].astype(jnp.float32)

    ap = jnp.abs(p)
    soft = jnp.log1p(jnp.exp(-ap))
    relu = 0.5 * (p + ap)
    loss_base = maskv * (relu + soft - p * t)

    @pl.when(jnp.logical_and(b == 0, c == 0))
    def _():
        loss_ref[0, 0] = 0.0
        cnt_ref[0, 0] = 0.0

    loss_ref[0, 0] += jnp.sum(loss_base)
    cnt_ref[0, 0] += jnp.sum(maskv)

    t_any = (jnp.any(t == 1.0) | jnp.any(tp_ref[0] == 1.0)
             | jnp.any(tn_ref[0] == 1.0))

    @pl.when(t_any)
    def _():
        T = (t == 1.0).astype(jnp.float32)
        cpk = 256.0 * T + maskv

        c_ref[pl.ds(1, ROWS), :] = cpk
        c_ref[pl.ds(0, 1), :] = (
            256.0 * (tp_ref[0] == 1.0).astype(jnp.float32)
            + mp_ref[0].astype(jnp.float32)) * pv
        c_ref[pl.ds(ROWS + 1, 1), :] = (
            256.0 * (tn_ref[0] == 1.0).astype(jnp.float32)
            + mn_ref[0].astype(jnp.float32)) * nv

        row_sum = c_ref[pl.ds(0, ROWS), :] + c_ref[pl.ds(2, ROWS), :]
        lane = jax.lax.broadcasted_iota(jnp.int32, (ROWS, SL), 1)
        right = jnp.where(lane >= L, pltpu.roll(cpk, L, 1), 0.0)
        left = jnp.where(lane < SL - L, pltpu.roll(cpk, SL - L, 1), 0.0)
        sc = row_sum + right + left

        A = jnp.floor(sc * (1.0 / 256.0))
        valid = sc - 256.0 * A
        D = A * maskv - T * valid
        loss_ref[0, 0] += -EPS4 * jnp.sum((p * maskv) * D)


@jax.jit
def kernel(predict, target, mask):
    p3 = predict.reshape(B, S, SL)
    t3 = target.reshape(B, S, SL)
    m3 = mask.reshape(B, S, SL)
    t_rows = t3.reshape(B * S, 1, SL)
    m_rows = m3.reshape(B * S, 1, SL)

    main = pl.BlockSpec((1, ROWS, SL), lambda b, c: (b, c, 0))
    prev = pl.BlockSpec(
        (1, 1, SL), lambda b, c: (b * S + jnp.maximum(c * ROWS - 1, 0), 0, 0))
    nxt = pl.BlockSpec(
        (1, 1, SL), lambda b, c: (b * S + jnp.minimum(c * ROWS + ROWS, S - 1), 0, 0))
    out = pl.BlockSpec(memory_space=pltpu.SMEM)

    loss, cnt = pl.pallas_call(
        _body,
        grid=(B, NC),
        in_specs=[main, main, main, prev, nxt, prev, nxt],
        out_specs=[out, out],
        out_shape=[
            jax.ShapeDtypeStruct((1, 1), jnp.float32),
            jax.ShapeDtypeStruct((1, 1), jnp.float32),
        ],
        scratch_shapes=[pltpu.VMEM((ROWS + 2, SL), jnp.float32)],
    )(p3, t3, m3, t_rows, t_rows, m_rows, m_rows)
    return loss[0, 0] / cnt[0, 0]
